# Initial kernel scaffold; baseline (speedup 1.0000x reference)
import jax
import jax.numpy as jnp
from jax.experimental import pallas as pl


def kernel(x, edge_index, batch, W2, b2, g3, be3, W3, b3, g4, be4, W4, b4, g5, be5, ew1, Wrel1, brel1, Wroot1, g6, be6, ew2, Wrel2, brel2, Wroot2, g7, be7, W5, b5, Whr, bhr, Whf, bhf, Wfa, bfa):
    raise NotImplementedError("write your pallas kernel here")


# speedup vs baseline: 3.6270x; 3.6270x over previous
import functools

import jax
import jax.numpy as jnp
from jax import lax
from jax.experimental import pallas as pl
from jax.experimental.pallas import tpu as pltpu

_B = 256
_NEL = 19
_N = _B * _NEL
_D0 = 20000
_WLEN = 25
_NPOOL = 800
_CHUNK = 3200
_NCHUNK = 6
_R = 152
_GRID = _N // _R


def _pool_mats():
    j = jnp.arange(_CHUNK)
    pc = (j[:, None] // _WLEN == jnp.arange(128)[None, :]).astype(jnp.float32) / _WLEN
    jt = jnp.arange(_D0 - _NCHUNK * _CHUNK)
    pt = (jt[:, None] // _WLEN == jnp.arange(128)[None, :]).astype(jnp.float32) / _WLEN
    return pc, pt


def _k1_body(x_ref, pc_ref, pt_ref, w2_ref, b2_ref, h1_ref, s_ref, q_ref, pooled_ref):
    for c in range(_NCHUNK):
        pooled_ref[:, 128 * c:128 * (c + 1)] = jnp.dot(
            x_ref[:, _CHUNK * c:_CHUNK * (c + 1)], pc_ref[...],
            preferred_element_type=jnp.float32)
    pooled_ref[:, _NCHUNK * 128:(_NCHUNK + 1) * 128] = jnp.dot(
        x_ref[:, _NCHUNK * _CHUNK:_D0], pt_ref[...],
        preferred_element_type=jnp.float32)
    h = jnp.dot(pooled_ref[...], w2_ref[...], preferred_element_type=jnp.float32)
    h = jnp.maximum(h + b2_ref[...], 0.0)
    h1_ref[...] = h
    h3 = h.reshape(_R // _NEL, _NEL, 512)
    ps = jnp.sum(h3, axis=(0, 2)).reshape(1, _NEL)
    pq = jnp.sum(h3 * h3, axis=(0, 2)).reshape(1, _NEL)

    @pl.when(pl.program_id(0) == 0)
    def _init():
        s_ref[...] = jnp.zeros_like(s_ref)
        q_ref[...] = jnp.zeros_like(q_ref)

    s_ref[...] += ps
    q_ref[...] += pq


def _stage1(x, w2pad, b2):
    pc, pt = _pool_mats()
    return pl.pallas_call(
        _k1_body,
        grid=(_GRID,),
        in_specs=[
            pl.BlockSpec((_R, _D0), lambda i: (i, 0)),
            pl.BlockSpec((_CHUNK, 128), lambda i: (0, 0)),
            pl.BlockSpec((_D0 - _NCHUNK * _CHUNK, 128), lambda i: (0, 0)),
            pl.BlockSpec(((_NCHUNK + 1) * 128, 512), lambda i: (0, 0)),
            pl.BlockSpec((1, 512), lambda i: (0, 0)),
        ],
        out_specs=[
            pl.BlockSpec((_R, 512), lambda i: (i, 0)),
            pl.BlockSpec((1, _NEL), lambda i: (0, 0)),
            pl.BlockSpec((1, _NEL), lambda i: (0, 0)),
        ],
        out_shape=[
            jax.ShapeDtypeStruct((_N, 512), jnp.float32),
            jax.ShapeDtypeStruct((1, _NEL), jnp.float32),
            jax.ShapeDtypeStruct((1, _NEL), jnp.float32),
        ],
        scratch_shapes=[pltpu.VMEM((_R, (_NCHUNK + 1) * 128), jnp.float32)],
    )(x, pc, pt, w2pad, b2)


def _bn_scale_shift(s, q, count, g, be):
    m = s / count
    v = q / count - m * m
    inv = lax.rsqrt(v + 1e-5) * g
    return inv, be - m * inv


def kernel(x, edge_index, batch, W2, b2, g3, be3, W3, b3, g4, be4, W4, b4,
           g5, be5, ew1, Wrel1, brel1, Wroot1, g6, be6, ew2, Wrel2, brel2,
           Wroot2, g7, be7, W5, b5, Whr, bhr, Whf, bhf, Wfa, bfa):
    w2pad = jnp.concatenate(
        [W2, jnp.zeros(((_NCHUNK + 1) * 128 - _NPOOL, 512), jnp.float32)], axis=0)
    h1, s1, q1 = _stage1(x, w2pad, b2.reshape(1, 512))

    inv1, sh1 = _bn_scale_shift(s1[0], q1[0], _B * 512.0, g3, be3)
    h = h1.reshape(_B, _NEL, 512) * inv1[None, :, None] + sh1[None, :, None]
    h = jax.nn.relu(h @ W3 + b3)
    m = h.mean(axis=(0, 2), keepdims=True)
    v = h.var(axis=(0, 2), keepdims=True)
    h = (h - m) / jnp.sqrt(v + 1e-5) * g4.reshape(1, -1, 1) + be4.reshape(1, -1, 1)
    h = jax.nn.relu(h @ W4 + b4)
    m = h.mean(axis=(0, 2), keepdims=True)
    v = h.var(axis=(0, 2), keepdims=True)
    h = (h - m) / jnp.sqrt(v + 1e-5) * g5.reshape(1, -1, 1) + be5.reshape(1, -1, 1)
    h = h.reshape(_N, 128)
    src = edge_index[0]
    dst = edge_index[1]

    def gconv(hh, ew, Wrel, brel, Wroot):
        w = jax.nn.softplus(ew)
        w_exp = jnp.tile(w, _B)
        msg = hh[src] * w_exp[:, None]
        agg = jnp.zeros_like(hh).at[dst].add(msg)
        return agg @ Wrel + brel + hh @ Wroot

    h = jax.nn.relu(gconv(h, ew1, Wrel1, brel1, Wroot1))
    mm = h.mean(axis=0)
    vv = h.var(axis=0)
    h = (h - mm) / jnp.sqrt(vv + 1e-5) * g6 + be6
    h = jax.nn.relu(gconv(h, ew2, Wrel2, brel2, Wroot2))
    mm = h.mean(axis=0)
    vv = h.var(axis=0)
    h = (h - mm) / jnp.sqrt(vv + 1e-5) * g7 + be7
    pooled = h.reshape(_B, _NEL, 64).max(axis=1)
    feat = jax.nn.relu(pooled @ W5 + b5)
    p0 = jax.nn.softmax(feat @ Whr + bhr, axis=1)
    p1 = jax.nn.softmax(feat @ Whf + bhf, axis=1)
    p2 = jax.nn.softmax(feat @ Wfa + bfa, axis=1)
    p_hc = p0[:, 0] * p1[:, 0]
    p_ad = p0[:, 1] * p2[:, 1]
    p_ftd = p0[:, 0] * p1[:, 1] + p0[:, 1] * p2[:, 0]
    final_prob = jnp.stack((p_hc, p_ftd, p_ad), axis=1)
    return jnp.log(final_prob + 1e-8)
